# Initial kernel scaffold; baseline (speedup 1.0000x reference)
import jax
import jax.numpy as jnp
from jax.experimental import pallas as pl


def kernel(edge_index, edge_weight, embedding, b):
    raise NotImplementedError("write your pallas kernel here")


# speedup vs baseline: 2.2255x; 2.2255x over previous
import functools

import jax
import jax.numpy as jnp
from jax import lax
from jax.experimental import pallas as pl
from jax.experimental.pallas import tpu as pltpu
from jax.experimental.pallas import tpu_sc as plsc

N = 50000
E = 800000
D = 100
DP = 128
NLAYERS = 3

NC = 2
NS = 16
LN = 16

NBUCK = 8
BN = N // NBUCK
BN_PAD = 6272
NP = NBUCK * BN_PAD
ROWS_PER_TILE = BN_PAD // NS
ZR = 56

EPT = E // NS
WIN = 10000
NWIN = EPT // WIN
GPW = WIN // LN
G = 32
STG = WIN + 2 * G

_mesh = plsc.VectorSubcoreMesh(core_axis_name="c", subcore_axis_name="s")


@functools.partial(
    pl.kernel,
    out_type=jax.ShapeDtypeStruct((NP, DP), jnp.float32),
    mesh=_mesh,
    compiler_params=pltpu.CompilerParams(needs_layout_passes=False),
    scratch_types=[
        pltpu.VMEM((WIN,), jnp.int32),
        pltpu.VMEM((WIN,), jnp.int32),
        pltpu.VMEM((WIN,), jnp.float32),
        pltpu.VMEM((STG,), jnp.int32),
        pltpu.VMEM((STG,), jnp.int32),
        pltpu.VMEM((STG,), jnp.float32),
        pltpu.VMEM((G, DP), jnp.float32),
        pltpu.VMEM((1, G), jnp.int32),
        pltpu.VMEM((ZR, DP), jnp.float32),
        pltpu.VMEM_SHARED((BN_PAD, DP), jnp.float32),
    ],
)
def _spmm_layer(dst_hbm, src_hbm, w_hbm, table_hbm, out_hbm,
                win_dst, win_src, win_w, st_src, st_dstl, st_w,
                rows, idx2d, zbuf, acc):
    c = lax.axis_index("c")
    s = lax.axis_index("s")
    zf = jnp.zeros((LN,), jnp.float32)
    zi = jnp.zeros((LN,), jnp.int32)
    lanes = lax.iota(jnp.int32, LN)

    for r in range(ZR):
        for q in range(DP // LN):
            zbuf[jnp.int32(r), pl.ds(q * LN, LN)] = zf

    ebase = s * jnp.int32(EPT)

    for b_local in range(NBUCK // NC):
        bucket = c * jnp.int32(NBUCK // NC) + jnp.int32(b_local)
        lo = bucket * jnp.int32(BN)

        for t in range(ROWS_PER_TILE // ZR):
            pltpu.sync_copy(
                zbuf,
                acc.at[pl.ds(s * jnp.int32(ROWS_PER_TILE) + jnp.int32(t * ZR),
                             ZR)])
        plsc.subcore_barrier()

        def window_body(wi, carry):
            wbase = ebase + wi * jnp.int32(WIN)
            pltpu.sync_copy(dst_hbm.at[pl.ds(wbase, WIN)], win_dst)
            pltpu.sync_copy(src_hbm.at[pl.ds(wbase, WIN)], win_src)
            pltpu.sync_copy(w_hbm.at[pl.ds(wbase, WIN)], win_w)

            def scan_body(g, off):
                gb = g * jnp.int32(LN)
                dv = win_dst[pl.ds(gb, LN)]
                sv = win_src[pl.ds(gb, LN)]
                wv = win_w[pl.ds(gb, LN)]
                sv = sv + (sv // jnp.int32(BN)) * jnp.int32(BN_PAD - BN)
                m = (dv >= lo) & (dv < lo + jnp.int32(BN))
                plsc.store_compressed(st_src.at[pl.ds(off, LN)], sv, mask=m)
                plsc.store_compressed(st_w.at[pl.ds(off, LN)], wv, mask=m)
                plsc.store_compressed(
                    st_dstl.at[pl.ds(off, LN)], dv - lo, mask=m)
                return off + jnp.sum(m.astype(jnp.int32), dtype=jnp.int32)

            n = lax.fori_loop(jnp.int32(0), jnp.int32(GPW), scan_body,
                              jnp.int32(0))

            for k in range(G // LN):
                st_src[pl.ds(n + jnp.int32(k * LN), LN)] = zi
                st_w[pl.ds(n + jnp.int32(k * LN), LN)] = zf
                st_dstl[pl.ds(n + jnp.int32(k * LN), LN)] = zi

            nb = (n + jnp.int32(G - 1)) // jnp.int32(G)

            def batch_body(j, bcarry):
                base = j * jnp.int32(G)
                for k in range(G // LN):
                    idx2d[jnp.int32(0), pl.ds(k * LN, LN)] = (
                        st_dstl[pl.ds(base + jnp.int32(k * LN), LN)])
                pltpu.sync_copy(table_hbm.at[st_src.at[pl.ds(base, G)]], rows)
                for k in range(G // LN):
                    wv = st_w[pl.ds(base + jnp.int32(k * LN), LN)]
                    for r in range(LN):
                        w_s = jnp.sum(jnp.where(lanes == r, wv, 0.0))
                        ri = k * LN + r
                        for q in range(DP // LN):
                            rows[jnp.int32(ri), pl.ds(q * LN, LN)] = (
                                rows[jnp.int32(ri), pl.ds(q * LN, LN)] * w_s)
                pltpu.sync_copy(rows, acc.at[idx2d.at[jnp.int32(0)]], add=True)
                return bcarry

            lax.fori_loop(jnp.int32(0), nb, batch_body, jnp.int32(0))
            return carry

        lax.fori_loop(jnp.int32(0), jnp.int32(NWIN), window_body,
                      jnp.int32(0))

        plsc.subcore_barrier()
        obase = bucket * jnp.int32(BN_PAD) + s * jnp.int32(ROWS_PER_TILE)
        pltpu.sync_copy(
            acc.at[pl.ds(s * jnp.int32(ROWS_PER_TILE), ROWS_PER_TILE)],
            out_hbm.at[pl.ds(obase, ROWS_PER_TILE)])
        plsc.subcore_barrier()


RB = 1024


def _final_body(b_ref, x0, x1, x2, x3, o_ref):
    o = jnp.zeros(o_ref.shape, jnp.float32)
    for i, xr in enumerate((x0, x1, x2, x3)):
        x = xr[...]
        ss = jnp.sum(x * x, axis=-1, keepdims=True)
        nrm = jnp.maximum(jnp.sqrt(ss), 1e-12)
        o = o + (b_ref[i] / nrm) * x
    o_ref[...] = o


_final_combine = pl.pallas_call(
    _final_body,
    grid=(NP // RB,),
    in_specs=[
        pl.BlockSpec(memory_space=pltpu.SMEM),
        pl.BlockSpec((RB, DP), lambda i: (i, 0)),
        pl.BlockSpec((RB, DP), lambda i: (i, 0)),
        pl.BlockSpec((RB, DP), lambda i: (i, 0)),
        pl.BlockSpec((RB, DP), lambda i: (i, 0)),
    ],
    out_specs=pl.BlockSpec((RB, DP), lambda i: (i, 0)),
    out_shape=jax.ShapeDtypeStruct((NP, DP), jnp.float32),
)


def kernel(edge_index, edge_weight, embedding, b):
    dst = edge_index[0].astype(jnp.int32)
    src = edge_index[1].astype(jnp.int32)
    w = edge_weight.astype(jnp.float32)
    emb_p = jnp.pad(embedding.astype(jnp.float32), ((0, 0), (0, DP - D)))
    t0 = jnp.pad(emb_p.reshape(NBUCK, BN, DP),
                 ((0, 0), (0, BN_PAD - BN), (0, 0))).reshape(NP, DP)
    t1 = _spmm_layer(dst, src, w, t0)
    t2 = _spmm_layer(dst, src, w, t1)
    t3 = _spmm_layer(dst, src, w, t2)
    bv = b.reshape(NLAYERS + 1).astype(jnp.float32)
    x64_was_on = jax.config.jax_enable_x64
    if x64_was_on:
        jax.config.update("jax_enable_x64", False)
    try:
        out = _final_combine(bv, t0, t1, t2, t3)
    finally:
        if x64_was_on:
            jax.config.update("jax_enable_x64", True)
    return out.reshape(NBUCK, BN_PAD, DP)[:, :BN, :D].reshape(N, D)


# speedup vs baseline: 2.6741x; 1.2016x over previous
import functools

import jax
import jax.numpy as jnp
from jax import lax
from jax.experimental import pallas as pl
from jax.experimental.pallas import tpu as pltpu
from jax.experimental.pallas import tpu_sc as plsc

N = 50000
E = 800000
D = 100
DP = 128
NLAYERS = 3

NC = 2
NS = 16
LN = 16

NBUCK = 8
BN = N // NBUCK
BN_PAD = 6272
NP = NBUCK * BN_PAD
ROWS_PER_TILE = BN_PAD // NS
ZR = 14

EPT = E // NS
WIN = 10000
NWIN = EPT // WIN
GPW = WIN // LN
G = 64
STG = WIN + G

_mesh = plsc.VectorSubcoreMesh(core_axis_name="c", subcore_axis_name="s")


@functools.partial(
    pl.kernel,
    out_type=jax.ShapeDtypeStruct((NP, DP), jnp.float32),
    mesh=_mesh,
    compiler_params=pltpu.CompilerParams(needs_layout_passes=False),
    scratch_types=[
        pltpu.VMEM((WIN,), jnp.int32),
        pltpu.VMEM((WIN,), jnp.int32),
        pltpu.VMEM((WIN,), jnp.float32),
        pltpu.VMEM((STG,), jnp.int32),
        pltpu.VMEM((STG,), jnp.int32),
        pltpu.VMEM((STG,), jnp.float32),
        pltpu.VMEM((G, DP), jnp.float32),
        pltpu.VMEM((G, DP), jnp.float32),
        pltpu.VMEM((2, G), jnp.int32),
        pltpu.VMEM((ZR, DP), jnp.float32),
        pltpu.VMEM_SHARED((BN_PAD, DP), jnp.float32),
        pltpu.SemaphoreType.DMA,
        pltpu.SemaphoreType.DMA,
        pltpu.SemaphoreType.DMA,
        pltpu.SemaphoreType.DMA,
    ],
)
def _spmm_layer(dst_hbm, src_hbm, w_hbm, table_hbm, out_hbm,
                win_dst, win_src, win_w, st_src, st_dstl, st_w,
                rows0, rows1, idx2d, zbuf, acc,
                sem_g0, sem_g1, sem_s0, sem_s1):
    c = lax.axis_index("c")
    s = lax.axis_index("s")
    zf = jnp.zeros((LN,), jnp.float32)
    zi = jnp.zeros((LN,), jnp.int32)
    lanes = lax.iota(jnp.int32, LN)

    for r in range(ZR):
        for q in range(DP // LN):
            zbuf[jnp.int32(r), pl.ds(q * LN, LN)] = zf

    ebase = s * jnp.int32(EPT)

    def bucket_body(b_local, bc):
        bucket = c * jnp.int32(NBUCK // NC) + b_local
        lo = bucket * jnp.int32(BN)

        for t in range(ROWS_PER_TILE // ZR):
            pltpu.sync_copy(
                zbuf,
                acc.at[pl.ds(s * jnp.int32(ROWS_PER_TILE) + jnp.int32(t * ZR),
                             ZR)])
        plsc.subcore_barrier()

        def window_body(wi, carry):
            wbase = ebase + wi * jnp.int32(WIN)
            pltpu.sync_copy(dst_hbm.at[pl.ds(wbase, WIN)], win_dst)
            pltpu.sync_copy(src_hbm.at[pl.ds(wbase, WIN)], win_src)
            pltpu.sync_copy(w_hbm.at[pl.ds(wbase, WIN)], win_w)

            def scan_body(g, off):
                gb = g * jnp.int32(LN)
                dv = win_dst[pl.ds(gb, LN)]
                sv = win_src[pl.ds(gb, LN)]
                wv = win_w[pl.ds(gb, LN)]
                sv = sv + (sv // jnp.int32(BN)) * jnp.int32(BN_PAD - BN)
                m = (dv >= lo) & (dv < lo + jnp.int32(BN))
                plsc.store_compressed(st_src.at[pl.ds(off, LN)], sv, mask=m)
                plsc.store_compressed(st_w.at[pl.ds(off, LN)], wv, mask=m)
                plsc.store_compressed(
                    st_dstl.at[pl.ds(off, LN)], dv - lo, mask=m)
                return off + jnp.sum(m.astype(jnp.int32), dtype=jnp.int32)

            n = lax.fori_loop(jnp.int32(0), jnp.int32(GPW), scan_body,
                              jnp.int32(0))

            for k in range(G // LN):
                st_src[pl.ds(n + jnp.int32(k * LN), LN)] = zi
                st_w[pl.ds(n + jnp.int32(k * LN), LN)] = zf
                st_dstl[pl.ds(n + jnp.int32(k * LN), LN)] = zi

            nb = (n + jnp.int32(G - 1)) // jnp.int32(G)

            bufs = ((rows0, sem_g0, sem_s0, jnp.int32(0)),
                    (rows1, sem_g1, sem_s1, jnp.int32(1)))

            def gather_start(j, rb, sg):
                pltpu.async_copy(
                    table_hbm.at[st_src.at[pl.ds(j * jnp.int32(G), G)]],
                    rb, sg)

            @pl.when(nb >= jnp.int32(1))
            def _prime():
                gather_start(jnp.int32(0), rows0, sem_g0)

            def batch_body(j, bcarry):
                par = j % jnp.int32(2)

                def half(mine, other):
                    rb, sg, ss, row = mine
                    rb_o, sg_o, ss_o, row_o = other
                    base = j * jnp.int32(G)
                    pltpu.make_async_copy(
                        table_hbm.at[st_src.at[pl.ds(base, G)]], rb, sg
                    ).wait()
                    @pl.when(j >= jnp.int32(1))
                    def _():
                        pltpu.make_async_copy(
                            rb_o, acc.at[idx2d.at[row_o]], ss_o).wait()
                    @pl.when(j + jnp.int32(1) < nb)
                    def _():
                        gather_start(j + jnp.int32(1), rb_o, sg_o)
                    for k in range(G // LN):
                        idx2d[row, pl.ds(k * LN, LN)] = (
                            st_dstl[pl.ds(base + jnp.int32(k * LN), LN)])
                    for k in range(G // LN):
                        wv = st_w[pl.ds(base + jnp.int32(k * LN), LN)]
                        for r in range(LN):
                            w_s = jnp.sum(jnp.where(lanes == r, wv, 0.0))
                            ri = jnp.int32(k * LN + r)
                            for q in range(DP // LN):
                                rb[ri, pl.ds(q * LN, LN)] = (
                                    rb[ri, pl.ds(q * LN, LN)] * w_s)
                    pltpu.async_copy(rb, acc.at[idx2d.at[row]], ss, add=True)

                @pl.when(par == jnp.int32(0))
                def _():
                    half(bufs[0], bufs[1])

                @pl.when(par == jnp.int32(1))
                def _():
                    half(bufs[1], bufs[0])

                return bcarry

            lax.fori_loop(jnp.int32(0), nb, batch_body, jnp.int32(0))

            @pl.when(nb >= jnp.int32(1))
            def _drain():
                lp = (nb - jnp.int32(1)) % jnp.int32(2)

                @pl.when(lp == jnp.int32(0))
                def _():
                    pltpu.make_async_copy(
                        rows0, acc.at[idx2d.at[jnp.int32(0)]], sem_s0).wait()

                @pl.when(lp == jnp.int32(1))
                def _():
                    pltpu.make_async_copy(
                        rows1, acc.at[idx2d.at[jnp.int32(1)]], sem_s1).wait()

            return carry

        lax.fori_loop(jnp.int32(0), jnp.int32(NWIN), window_body,
                      jnp.int32(0))

        plsc.subcore_barrier()
        obase = bucket * jnp.int32(BN_PAD) + s * jnp.int32(ROWS_PER_TILE)
        pltpu.sync_copy(
            acc.at[pl.ds(s * jnp.int32(ROWS_PER_TILE), ROWS_PER_TILE)],
            out_hbm.at[pl.ds(obase, ROWS_PER_TILE)])
        plsc.subcore_barrier()
        return bc

    lax.fori_loop(jnp.int32(0), jnp.int32(NBUCK // NC), bucket_body,
                  jnp.int32(0))


RB = 1024


def _final_body(b_ref, x0, x1, x2, x3, o_ref):
    o = jnp.zeros(o_ref.shape, jnp.float32)
    for i, xr in enumerate((x0, x1, x2, x3)):
        x = xr[...]
        ss = jnp.sum(x * x, axis=-1, keepdims=True)
        nrm = jnp.maximum(jnp.sqrt(ss), 1e-12)
        o = o + (b_ref[i] / nrm) * x
    o_ref[...] = o


_final_combine = pl.pallas_call(
    _final_body,
    grid=(NP // RB,),
    in_specs=[
        pl.BlockSpec(memory_space=pltpu.SMEM),
        pl.BlockSpec((RB, DP), lambda i: (i, 0)),
        pl.BlockSpec((RB, DP), lambda i: (i, 0)),
        pl.BlockSpec((RB, DP), lambda i: (i, 0)),
        pl.BlockSpec((RB, DP), lambda i: (i, 0)),
    ],
    out_specs=pl.BlockSpec((RB, DP), lambda i: (i, 0)),
    out_shape=jax.ShapeDtypeStruct((NP, DP), jnp.float32),
)


def kernel(edge_index, edge_weight, embedding, b):
    dst = edge_index[0].astype(jnp.int32)
    src = edge_index[1].astype(jnp.int32)
    w = edge_weight.astype(jnp.float32)
    emb_p = jnp.pad(embedding.astype(jnp.float32), ((0, 0), (0, DP - D)))
    t0 = jnp.pad(emb_p.reshape(NBUCK, BN, DP),
                 ((0, 0), (0, BN_PAD - BN), (0, 0))).reshape(NP, DP)
    t1 = _spmm_layer(dst, src, w, t0)
    t2 = _spmm_layer(dst, src, w, t1)
    t3 = _spmm_layer(dst, src, w, t2)
    bv = b.reshape(NLAYERS + 1).astype(jnp.float32)
    x64_was_on = jax.config.jax_enable_x64
    if x64_was_on:
        jax.config.update("jax_enable_x64", False)
    try:
        out = _final_combine(bv, t0, t1, t2, t3)
    finally:
        if x64_was_on:
            jax.config.update("jax_enable_x64", True)
    return out.reshape(NBUCK, BN_PAD, DP)[:, :BN, :D].reshape(N, D)


# speedup vs baseline: 2.7327x; 1.0219x over previous
import functools

import jax
import jax.numpy as jnp
from jax import lax
from jax.experimental import pallas as pl
from jax.experimental.pallas import tpu as pltpu
from jax.experimental.pallas import tpu_sc as plsc

N = 50000
E = 800000
D = 100
DP = 128
NLAYERS = 3

NC = 2
NS = 16
LN = 16

NBUCK = 8
BN = N // NBUCK
BN_PAD = 6272
NP = NBUCK * BN_PAD
ROWS_PER_TILE = BN_PAD // NS
ZR = 14

EPT = E // NS
WIN = 10000
NWIN = EPT // WIN
GPW = WIN // LN
UNR = 5
G = 64
STG = WIN + G

_mesh = plsc.VectorSubcoreMesh(core_axis_name="c", subcore_axis_name="s")


@functools.partial(
    pl.kernel,
    out_type=jax.ShapeDtypeStruct((NP, DP), jnp.float32),
    mesh=_mesh,
    compiler_params=pltpu.CompilerParams(needs_layout_passes=False),
    scratch_types=[
        pltpu.VMEM((WIN,), jnp.int32),
        pltpu.VMEM((WIN,), jnp.int32),
        pltpu.VMEM((WIN,), jnp.float32),
        pltpu.VMEM((STG,), jnp.int32),
        pltpu.VMEM((STG,), jnp.int32),
        pltpu.VMEM((STG,), jnp.float32),
        pltpu.VMEM((G, DP), jnp.float32),
        pltpu.VMEM((G, DP), jnp.float32),
        pltpu.VMEM((2, G), jnp.int32),
        pltpu.VMEM((ZR, DP), jnp.float32),
        pltpu.VMEM_SHARED((BN_PAD, DP), jnp.float32),
        pltpu.SemaphoreType.DMA,
        pltpu.SemaphoreType.DMA,
        pltpu.SemaphoreType.DMA,
        pltpu.SemaphoreType.DMA,
    ],
)
def _spmm_layer(dst_hbm, src_hbm, w_hbm, table_hbm, out_hbm,
                win_dst, win_src, win_w, st_src, st_dstl, st_w,
                rows0, rows1, idx2d, zbuf, acc,
                sem_g0, sem_g1, sem_s0, sem_s1):
    c = lax.axis_index("c")
    s = lax.axis_index("s")
    zf = jnp.zeros((LN,), jnp.float32)
    zi = jnp.zeros((LN,), jnp.int32)
    lanes = lax.iota(jnp.int32, LN)

    for r in range(ZR):
        for q in range(DP // LN):
            zbuf[jnp.int32(r), pl.ds(q * LN, LN)] = zf

    ebase = s * jnp.int32(EPT)

    def bucket_body(b_local, bc):
        bucket = c * jnp.int32(NBUCK // NC) + b_local
        lo = bucket * jnp.int32(BN)

        for t in range(ROWS_PER_TILE // ZR):
            pltpu.sync_copy(
                zbuf,
                acc.at[pl.ds(s * jnp.int32(ROWS_PER_TILE) + jnp.int32(t * ZR),
                             ZR)])
        plsc.subcore_barrier()

        def window_body(wi, carry):
            wbase = ebase + wi * jnp.int32(WIN)
            pltpu.sync_copy(dst_hbm.at[pl.ds(wbase, WIN)], win_dst)
            pltpu.sync_copy(src_hbm.at[pl.ds(wbase, WIN)], win_src)
            pltpu.sync_copy(w_hbm.at[pl.ds(wbase, WIN)], win_w)

            def scan_body(g, off):
                gb = g * jnp.int32(UNR * LN)
                grp = []
                for u in range(UNR):
                    ub = gb + jnp.int32(u * LN)
                    dv = win_dst[pl.ds(ub, LN)]
                    sv = win_src[pl.ds(ub, LN)]
                    wv = win_w[pl.ds(ub, LN)]
                    sv = sv + (sv // jnp.int32(BN)) * jnp.int32(BN_PAD - BN)
                    m = (dv >= lo) & (dv < lo + jnp.int32(BN))
                    cnt = plsc.all_reduce_population_count(m)[0]
                    grp.append((dv, sv, wv, m, cnt))
                for dv, sv, wv, m, cnt in grp:
                    plsc.store_compressed(st_src.at[pl.ds(off, LN)], sv,
                                          mask=m)
                    plsc.store_compressed(st_w.at[pl.ds(off, LN)], wv, mask=m)
                    plsc.store_compressed(
                        st_dstl.at[pl.ds(off, LN)], dv - lo, mask=m)
                    off = off + cnt
                return off

            n = lax.fori_loop(jnp.int32(0), jnp.int32(GPW // UNR), scan_body,
                              jnp.int32(0))

            for k in range(G // LN):
                st_src[pl.ds(n + jnp.int32(k * LN), LN)] = zi
                st_w[pl.ds(n + jnp.int32(k * LN), LN)] = zf
                st_dstl[pl.ds(n + jnp.int32(k * LN), LN)] = zi

            nb = (n + jnp.int32(G - 1)) // jnp.int32(G)

            bufs = ((rows0, sem_g0, sem_s0, jnp.int32(0)),
                    (rows1, sem_g1, sem_s1, jnp.int32(1)))

            def gather_start(j, rb, sg):
                pltpu.async_copy(
                    table_hbm.at[st_src.at[pl.ds(j * jnp.int32(G), G)]],
                    rb, sg)

            @pl.when(nb >= jnp.int32(1))
            def _prime():
                gather_start(jnp.int32(0), rows0, sem_g0)

            def batch_body(j, bcarry):
                par = j % jnp.int32(2)

                def half(mine, other):
                    rb, sg, ss, row = mine
                    rb_o, sg_o, ss_o, row_o = other
                    base = j * jnp.int32(G)
                    pltpu.make_async_copy(
                        table_hbm.at[st_src.at[pl.ds(base, G)]], rb, sg
                    ).wait()
                    @pl.when(j >= jnp.int32(1))
                    def _():
                        pltpu.make_async_copy(
                            rb_o, acc.at[idx2d.at[row_o]], ss_o).wait()
                    @pl.when(j + jnp.int32(1) < nb)
                    def _():
                        gather_start(j + jnp.int32(1), rb_o, sg_o)
                    for k in range(G // LN):
                        idx2d[row, pl.ds(k * LN, LN)] = (
                            st_dstl[pl.ds(base + jnp.int32(k * LN), LN)])
                    for k in range(G // LN):
                        wv = st_w[pl.ds(base + jnp.int32(k * LN), LN)]
                        for r in range(LN):
                            w_s = jnp.sum(jnp.where(lanes == r, wv, 0.0))
                            ri = jnp.int32(k * LN + r)
                            for q in range(DP // LN):
                                rb[ri, pl.ds(q * LN, LN)] = (
                                    rb[ri, pl.ds(q * LN, LN)] * w_s)
                    pltpu.async_copy(rb, acc.at[idx2d.at[row]], ss, add=True)

                @pl.when(par == jnp.int32(0))
                def _():
                    half(bufs[0], bufs[1])

                @pl.when(par == jnp.int32(1))
                def _():
                    half(bufs[1], bufs[0])

                return bcarry

            lax.fori_loop(jnp.int32(0), nb, batch_body, jnp.int32(0))

            @pl.when(nb >= jnp.int32(1))
            def _drain():
                lp = (nb - jnp.int32(1)) % jnp.int32(2)

                @pl.when(lp == jnp.int32(0))
                def _():
                    pltpu.make_async_copy(
                        rows0, acc.at[idx2d.at[jnp.int32(0)]], sem_s0).wait()

                @pl.when(lp == jnp.int32(1))
                def _():
                    pltpu.make_async_copy(
                        rows1, acc.at[idx2d.at[jnp.int32(1)]], sem_s1).wait()

            return carry

        lax.fori_loop(jnp.int32(0), jnp.int32(NWIN), window_body,
                      jnp.int32(0))

        plsc.subcore_barrier()
        obase = bucket * jnp.int32(BN_PAD) + s * jnp.int32(ROWS_PER_TILE)
        pltpu.sync_copy(
            acc.at[pl.ds(s * jnp.int32(ROWS_PER_TILE), ROWS_PER_TILE)],
            out_hbm.at[pl.ds(obase, ROWS_PER_TILE)])
        plsc.subcore_barrier()
        return bc

    lax.fori_loop(jnp.int32(0), jnp.int32(NBUCK // NC), bucket_body,
                  jnp.int32(0))


RB = 1024


def _final_body(b_ref, x0, x1, x2, x3, o_ref):
    o = jnp.zeros(o_ref.shape, jnp.float32)
    for i, xr in enumerate((x0, x1, x2, x3)):
        x = xr[...]
        ss = jnp.sum(x * x, axis=-1, keepdims=True)
        nrm = jnp.maximum(jnp.sqrt(ss), 1e-12)
        o = o + (b_ref[i] / nrm) * x
    o_ref[...] = o


_final_combine = pl.pallas_call(
    _final_body,
    grid=(NP // RB,),
    in_specs=[
        pl.BlockSpec(memory_space=pltpu.SMEM),
        pl.BlockSpec((RB, DP), lambda i: (i, 0)),
        pl.BlockSpec((RB, DP), lambda i: (i, 0)),
        pl.BlockSpec((RB, DP), lambda i: (i, 0)),
        pl.BlockSpec((RB, DP), lambda i: (i, 0)),
    ],
    out_specs=pl.BlockSpec((RB, DP), lambda i: (i, 0)),
    out_shape=jax.ShapeDtypeStruct((NP, DP), jnp.float32),
)


def kernel(edge_index, edge_weight, embedding, b):
    dst = edge_index[0].astype(jnp.int32)
    src = edge_index[1].astype(jnp.int32)
    w = edge_weight.astype(jnp.float32)
    emb_p = jnp.pad(embedding.astype(jnp.float32), ((0, 0), (0, DP - D)))
    t0 = jnp.pad(emb_p.reshape(NBUCK, BN, DP),
                 ((0, 0), (0, BN_PAD - BN), (0, 0))).reshape(NP, DP)
    t1 = _spmm_layer(dst, src, w, t0)
    t2 = _spmm_layer(dst, src, w, t1)
    t3 = _spmm_layer(dst, src, w, t2)
    bv = b.reshape(NLAYERS + 1).astype(jnp.float32)
    x64_was_on = jax.config.jax_enable_x64
    if x64_was_on:
        jax.config.update("jax_enable_x64", False)
    try:
        out = _final_combine(bv, t0, t1, t2, t3)
    finally:
        if x64_was_on:
            jax.config.update("jax_enable_x64", True)
    return out.reshape(NBUCK, BN_PAD, DP)[:, :BN, :D].reshape(N, D)


# speedup vs baseline: 2.8592x; 1.0463x over previous
import functools

import jax
import jax.numpy as jnp
from jax import lax
from jax.experimental import pallas as pl
from jax.experimental.pallas import tpu as pltpu
from jax.experimental.pallas import tpu_sc as plsc

N = 50000
E = 800000
D = 100
DP = 128
NLAYERS = 3

NC = 2
NS = 16
LN = 16

NBUCK = 8
BN = N // NBUCK
BN_PAD = 6272
NP = NBUCK * BN_PAD
ROWS_PER_TILE = BN_PAD // NS
ZR = 14

EPT = E // NS
WIN = 10000
NWIN = EPT // WIN
GPW = WIN // LN
UNR = 5
G = 64
STG = WIN + G

_mesh = plsc.VectorSubcoreMesh(core_axis_name="c", subcore_axis_name="s")


@functools.partial(
    pl.kernel,
    out_type=jax.ShapeDtypeStruct((NP, DP), jnp.float32),
    mesh=_mesh,
    compiler_params=pltpu.CompilerParams(needs_layout_passes=False),
    scratch_types=[
        pltpu.VMEM((WIN,), jnp.int32),
        pltpu.VMEM((WIN,), jnp.int32),
        pltpu.VMEM((WIN,), jnp.float32),
        pltpu.VMEM((STG,), jnp.int32),
        pltpu.VMEM((STG,), jnp.int32),
        pltpu.VMEM((STG,), jnp.float32),
        pltpu.VMEM((G, DP), jnp.float32),
        pltpu.VMEM((G, DP), jnp.float32),
        pltpu.VMEM((2, G), jnp.int32),
        pltpu.VMEM((ZR, DP), jnp.float32),
        pltpu.VMEM_SHARED((BN_PAD, DP), jnp.float32),
        pltpu.SemaphoreType.DMA,
        pltpu.SemaphoreType.DMA,
        pltpu.SemaphoreType.DMA,
        pltpu.SemaphoreType.DMA,
    ],
)
def _spmm_layer(dst_hbm, src_hbm, w_hbm, table_hbm, out_hbm,
                win_dst, win_src, win_w, st_src, st_dstl, st_w,
                rows0, rows1, idx2d, zbuf, acc,
                sem_g0, sem_g1, sem_s0, sem_s1):
    c = lax.axis_index("c")
    s = lax.axis_index("s")
    zf = jnp.zeros((LN,), jnp.float32)
    zi = jnp.zeros((LN,), jnp.int32)
    lanes = lax.iota(jnp.int32, LN)

    for r in range(ZR):
        for q in range(DP // LN):
            zbuf[jnp.int32(r), pl.ds(q * LN, LN)] = zf

    ebase = s * jnp.int32(EPT)

    def bucket_body(b_local, bc):
        bucket = c * jnp.int32(NBUCK // NC) + b_local
        lo = bucket * jnp.int32(BN)

        for t in range(ROWS_PER_TILE // ZR):
            pltpu.sync_copy(
                zbuf,
                acc.at[pl.ds(s * jnp.int32(ROWS_PER_TILE) + jnp.int32(t * ZR),
                             ZR)])
        plsc.subcore_barrier()

        def window_body(wi, carry):
            wbase = ebase + wi * jnp.int32(WIN)
            c1 = pltpu.async_copy(dst_hbm.at[pl.ds(wbase, WIN)], win_dst,
                                  sem_g0)
            c2 = pltpu.async_copy(src_hbm.at[pl.ds(wbase, WIN)], win_src,
                                  sem_g1)
            c3 = pltpu.async_copy(w_hbm.at[pl.ds(wbase, WIN)], win_w, sem_s0)
            c1.wait()
            c2.wait()
            c3.wait()

            def scan_body(g, off):
                gb = g * jnp.int32(UNR * LN)
                grp = []
                for u in range(UNR):
                    ub = gb + jnp.int32(u * LN)
                    dv = win_dst[pl.ds(ub, LN)]
                    sv = win_src[pl.ds(ub, LN)]
                    wv = win_w[pl.ds(ub, LN)]
                    sv = sv + (sv // jnp.int32(BN)) * jnp.int32(BN_PAD - BN)
                    m = (dv >= lo) & (dv < lo + jnp.int32(BN))
                    cnt = plsc.all_reduce_population_count(m)[0]
                    grp.append((dv, sv, wv, m, cnt))
                for dv, sv, wv, m, cnt in grp:
                    plsc.store_compressed(st_src.at[pl.ds(off, LN)], sv,
                                          mask=m)
                    plsc.store_compressed(st_w.at[pl.ds(off, LN)], wv, mask=m)
                    plsc.store_compressed(
                        st_dstl.at[pl.ds(off, LN)], dv - lo, mask=m)
                    off = off + cnt
                return off

            n = lax.fori_loop(jnp.int32(0), jnp.int32(GPW // UNR), scan_body,
                              jnp.int32(0))

            for k in range(G // LN):
                st_src[pl.ds(n + jnp.int32(k * LN), LN)] = zi
                st_w[pl.ds(n + jnp.int32(k * LN), LN)] = zf
                st_dstl[pl.ds(n + jnp.int32(k * LN), LN)] = zi

            nb = (n + jnp.int32(G - 1)) // jnp.int32(G)

            bufs = ((rows0, sem_g0, sem_s0, jnp.int32(0)),
                    (rows1, sem_g1, sem_s1, jnp.int32(1)))

            def gather_start(j, rb, sg):
                pltpu.async_copy(
                    table_hbm.at[st_src.at[pl.ds(j * jnp.int32(G), G)]],
                    rb, sg)

            @pl.when(nb >= jnp.int32(1))
            def _prime():
                gather_start(jnp.int32(0), rows0, sem_g0)

            def batch_body(j, bcarry):
                par = j % jnp.int32(2)

                def half(mine, other):
                    rb, sg, ss, row = mine
                    rb_o, sg_o, ss_o, row_o = other
                    base = j * jnp.int32(G)
                    pltpu.make_async_copy(
                        table_hbm.at[st_src.at[pl.ds(base, G)]], rb, sg
                    ).wait()
                    @pl.when(j >= jnp.int32(1))
                    def _():
                        pltpu.make_async_copy(
                            rb_o, acc.at[idx2d.at[row_o]], ss_o).wait()
                    @pl.when(j + jnp.int32(1) < nb)
                    def _():
                        gather_start(j + jnp.int32(1), rb_o, sg_o)
                    for k in range(G // LN):
                        idx2d[row, pl.ds(k * LN, LN)] = (
                            st_dstl[pl.ds(base + jnp.int32(k * LN), LN)])
                    for k in range(G // LN):
                        wv = st_w[pl.ds(base + jnp.int32(k * LN), LN)]
                        for r in range(LN):
                            w_s = jnp.sum(jnp.where(lanes == r, wv, 0.0))
                            ri = jnp.int32(k * LN + r)
                            for q in range(DP // LN):
                                rb[ri, pl.ds(q * LN, LN)] = (
                                    rb[ri, pl.ds(q * LN, LN)] * w_s)
                    pltpu.async_copy(rb, acc.at[idx2d.at[row]], ss, add=True)

                @pl.when(par == jnp.int32(0))
                def _():
                    half(bufs[0], bufs[1])

                @pl.when(par == jnp.int32(1))
                def _():
                    half(bufs[1], bufs[0])

                return bcarry

            lax.fori_loop(jnp.int32(0), nb, batch_body, jnp.int32(0))

            @pl.when(nb >= jnp.int32(1))
            def _drain():
                lp = (nb - jnp.int32(1)) % jnp.int32(2)

                @pl.when(lp == jnp.int32(0))
                def _():
                    pltpu.make_async_copy(
                        rows0, acc.at[idx2d.at[jnp.int32(0)]], sem_s0).wait()

                @pl.when(lp == jnp.int32(1))
                def _():
                    pltpu.make_async_copy(
                        rows1, acc.at[idx2d.at[jnp.int32(1)]], sem_s1).wait()

            return carry

        lax.fori_loop(jnp.int32(0), jnp.int32(NWIN), window_body,
                      jnp.int32(0))

        plsc.subcore_barrier()
        obase = bucket * jnp.int32(BN_PAD) + s * jnp.int32(ROWS_PER_TILE)
        pltpu.sync_copy(
            acc.at[pl.ds(s * jnp.int32(ROWS_PER_TILE), ROWS_PER_TILE)],
            out_hbm.at[pl.ds(obase, ROWS_PER_TILE)])
        plsc.subcore_barrier()
        return bc

    lax.fori_loop(jnp.int32(0), jnp.int32(NBUCK // NC), bucket_body,
                  jnp.int32(0))


RB = 1024


def _final_body(b_ref, x0, x1, x2, x3, o_ref):
    o = jnp.zeros(o_ref.shape, jnp.float32)
    for i, xr in enumerate((x0, x1, x2, x3)):
        x = xr[...]
        ss = jnp.sum(x * x, axis=-1, keepdims=True)
        nrm = jnp.maximum(jnp.sqrt(ss), 1e-12)
        o = o + (b_ref[i] / nrm) * x
    o_ref[...] = o


_final_combine = pl.pallas_call(
    _final_body,
    grid=(NP // RB,),
    in_specs=[
        pl.BlockSpec(memory_space=pltpu.SMEM),
        pl.BlockSpec((RB, DP), lambda i: (i, 0)),
        pl.BlockSpec((RB, DP), lambda i: (i, 0)),
        pl.BlockSpec((RB, DP), lambda i: (i, 0)),
        pl.BlockSpec((RB, DP), lambda i: (i, 0)),
    ],
    out_specs=pl.BlockSpec((RB, DP), lambda i: (i, 0)),
    out_shape=jax.ShapeDtypeStruct((NP, DP), jnp.float32),
)


def kernel(edge_index, edge_weight, embedding, b):
    dst = edge_index[0].astype(jnp.int32)
    src = edge_index[1].astype(jnp.int32)
    w = edge_weight.astype(jnp.float32)
    emb_p = jnp.pad(embedding.astype(jnp.float32), ((0, 0), (0, DP - D)))
    t0 = jnp.pad(emb_p.reshape(NBUCK, BN, DP),
                 ((0, 0), (0, BN_PAD - BN), (0, 0))).reshape(NP, DP)
    t1 = _spmm_layer(dst, src, w, t0)
    t2 = _spmm_layer(dst, src, w, t1)
    t3 = _spmm_layer(dst, src, w, t2)
    bv = b.reshape(NLAYERS + 1).astype(jnp.float32)
    x64_was_on = jax.config.jax_enable_x64
    if x64_was_on:
        jax.config.update("jax_enable_x64", False)
    try:
        out = _final_combine(bv, t0, t1, t2, t3)
    finally:
        if x64_was_on:
            jax.config.update("jax_enable_x64", True)
    return out.reshape(NBUCK, BN_PAD, DP)[:, :BN, :D].reshape(N, D)
